# baseline (speedup 1.0000x reference)
import jax
import jax.numpy as jnp
from jax import lax
from jax.experimental import pallas as pl
from jax.experimental.pallas import tpu as pltpu, tpu_sc as plsc

DIM = 64
VOCAB = 1000000
BATCH = 16384

_info = plsc.get_sparse_core_info()
_NC, _NS, _L = _info.num_cores, _info.num_subcores, _info.num_lanes
_NW = _NC * _NS
_BPW = BATCH // _NW
_HALF = _BPW // 2

_CW = 256
_VPW = 31232
_NCH = _VPW // _CW
_TAIL0 = _NW * _VPW
_XCH = (VOCAB - _TAIL0) // _CW
_T64 = _TAIL0 + _XCH * _CW


def _transpose_chunk(src_v, dst_v, width, lane):
    def col(j, carry):
        for k in range(DIM // _L):
            v = plsc.load_gather(src_v, [k * _L + lane,
                                         jnp.zeros((_L,), jnp.int32) + j])
            dst_v[j, pl.ds(k * _L, _L)] = v
        return carry

    lax.fori_loop(0, width, col, 0)


def _relayout_body(embt_hbm, outt_hbm, taile_hbm, tailo_hbm, rme_hbm, rmo_hbm,
                   ine_v, oute_v, ino_v, outo_v,
                   sie, soe, sio, soo):
    wid = lax.axis_index("s") * _NC + lax.axis_index("c")
    v0 = wid * _VPW
    nch = jnp.where(wid == _NW - 1, _NCH + _XCH, _NCH)
    lane = lax.iota(jnp.int32, _L)

    def off(c):
        return v0 + c * _CW

    pltpu.async_copy(embt_hbm.at[:, pl.ds(off(0), _CW)], ine_v, sie)
    pltpu.async_copy(outt_hbm.at[:, pl.ds(off(0), _CW)], ino_v, sio)

    def step(c, carry):
        o = off(c)
        pltpu.make_async_copy(embt_hbm.at[:, pl.ds(0, _CW)], ine_v, sie).wait()

        @pl.when(c > 0)
        def _():
            pltpu.make_async_copy(oute_v, rme_hbm.at[pl.ds(0, _CW), :],
                                  soe).wait()

        _transpose_chunk(ine_v, oute_v, _CW, lane)
        pltpu.async_copy(oute_v, rme_hbm.at[pl.ds(o, _CW), :], soe)

        @pl.when(c + 1 < nch)
        def _():
            pltpu.async_copy(embt_hbm.at[:, pl.ds(off(c + 1), _CW)],
                             ine_v, sie)

        pltpu.make_async_copy(outt_hbm.at[:, pl.ds(0, _CW)], ino_v, sio).wait()

        @pl.when(c > 0)
        def _():
            pltpu.make_async_copy(outo_v, rmo_hbm.at[pl.ds(0, _CW), :],
                                  soo).wait()

        _transpose_chunk(ino_v, outo_v, _CW, lane)
        pltpu.async_copy(outo_v, rmo_hbm.at[pl.ds(o, _CW), :], soo)

        @pl.when(c + 1 < nch)
        def _():
            pltpu.async_copy(outt_hbm.at[:, pl.ds(off(c + 1), _CW)],
                             ino_v, sio)

        return carry

    lax.fori_loop(0, nch, step, 0)
    pltpu.make_async_copy(oute_v, rme_hbm.at[pl.ds(0, _CW), :], soe).wait()
    pltpu.make_async_copy(outo_v, rmo_hbm.at[pl.ds(0, _CW), :], soo).wait()

    @pl.when(wid == _NW - 1)
    def _():
        pltpu.sync_copy(taile_hbm, oute_v.at[pl.ds(0, DIM), :])
        pltpu.sync_copy(oute_v.at[pl.ds(0, DIM), :],
                        rme_hbm.at[pl.ds(_T64, DIM), :])
        pltpu.sync_copy(tailo_hbm, outo_v.at[pl.ds(0, DIM), :])
        pltpu.sync_copy(outo_v.at[pl.ds(0, DIM), :],
                        rmo_hbm.at[pl.ds(_T64, DIM), :])


def _gather_body(target_hbm, context_hbm, emb_hbm, outtab_hbm, out_hbm,
                 tidx_s, cidx_s, trows_v, crows_v, res_v, sem):
    wid = lax.axis_index("s") * _NC + lax.axis_index("c")
    base = wid * _BPW

    pltpu.sync_copy(target_hbm.at[pl.ds(base, _BPW)], tidx_s)
    pltpu.sync_copy(context_hbm.at[pl.ds(base, _BPW)], cidx_s)

    lane = lax.iota(jnp.int32, _L)

    for h in range(2):
        off = h * _HALF

        def fire(b, carry):
            tvec = tidx_s[pl.ds(off + b * _L, _L)]
            cvec = cidx_s[pl.ds(off + b * _L, _L)]
            for l in range(_L):
                it = tvec[l]
                ic = cvec[l]
                j = b * _L + l
                pltpu.async_copy(emb_hbm.at[pl.ds(it, 1), :],
                                 trows_v.at[pl.ds(j, 1), :], sem)
                pltpu.async_copy(outtab_hbm.at[pl.ds(ic, 1), :],
                                 crows_v.at[pl.ds(j, 1), :], sem)
            return carry

        lax.fori_loop(0, _HALF // _L, fire, 0)

        def drain(j, carry):
            pltpu.make_async_copy(emb_hbm.at[pl.ds(0, 1), :],
                                  trows_v.at[pl.ds(0, 1), :], sem).wait()
            pltpu.make_async_copy(outtab_hbm.at[pl.ds(0, 1), :],
                                  crows_v.at[pl.ds(0, 1), :], sem).wait()
            return carry

        lax.fori_loop(0, _HALF, drain, 0)

        def group(g, carry):
            row = g * _L + lane
            acc = jnp.zeros((_L,), jnp.float32)
            for d in range(DIM):
                col = jnp.full((_L,), d, jnp.int32)
                tv = plsc.load_gather(trows_v, [row, col])
                cv = plsc.load_gather(crows_v, [row, col])
                acc = acc + tv * cv
            res_v[pl.ds(off + g * _L, _L)] = acc
            return carry

        lax.fori_loop(0, _HALF // _L, group, 0)

    pltpu.sync_copy(res_v, out_hbm.at[pl.ds(base, _BPW)])


def kernel(target, context, embeddings, output):
    mesh = plsc.VectorSubcoreMesh(core_axis_name="c", subcore_axis_name="s")
    relayout = pl.kernel(
        _relayout_body,
        out_type=(jax.ShapeDtypeStruct((VOCAB, DIM), jnp.float32),
                  jax.ShapeDtypeStruct((VOCAB, DIM), jnp.float32)),
        mesh=mesh,
        scratch_types=[
            pltpu.VMEM((DIM, _CW), jnp.float32),
            pltpu.VMEM((_CW, DIM), jnp.float32),
            pltpu.VMEM((DIM, _CW), jnp.float32),
            pltpu.VMEM((_CW, DIM), jnp.float32),
            pltpu.SemaphoreType.DMA,
            pltpu.SemaphoreType.DMA,
            pltpu.SemaphoreType.DMA,
            pltpu.SemaphoreType.DMA,
        ],
        compiler_params=pltpu.CompilerParams(needs_layout_passes=False),
    )
    gather = pl.kernel(
        _gather_body,
        out_type=jax.ShapeDtypeStruct((BATCH,), jnp.float32),
        mesh=mesh,
        scratch_types=[
            pltpu.VMEM((_BPW,), jnp.int32),
            pltpu.VMEM((_BPW,), jnp.int32),
            pltpu.VMEM((_HALF, DIM), jnp.float32),
            pltpu.VMEM((_HALF, DIM), jnp.float32),
            pltpu.VMEM((_BPW,), jnp.float32),
            pltpu.SemaphoreType.DMA,
        ],
        compiler_params=pltpu.CompilerParams(needs_layout_passes=False),
    )
    taile = jax.lax.slice(embeddings, (_T64, 0), (VOCAB, DIM))
    tailo = jax.lax.slice(output, (_T64, 0), (VOCAB, DIM))
    rme, rmo = relayout(jnp.swapaxes(embeddings, 0, 1),
                        jnp.swapaxes(output, 0, 1), taile, tailo)
    return gather(target.astype(jnp.int32), context.astype(jnp.int32),
                  rme, rmo)


# speedup vs baseline: 4.2481x; 4.2481x over previous
import jax
import jax.numpy as jnp
from jax import lax
from jax.experimental import pallas as pl
from jax.experimental.pallas import tpu as pltpu, tpu_sc as plsc

DIM = 64
VOCAB = 1000000
BATCH = 16384

_info = plsc.get_sparse_core_info()
_NC, _NS, _L = _info.num_cores, _info.num_subcores, _info.num_lanes
_NW = _NC * _NS
_BPW = BATCH // _NW
_HALF = _BPW // 2


def _body(target_hbm, context_hbm, emb_hbm, outtab_hbm, out_hbm,
          tidx_s, cidx_s, trows_v, crows_v, res_v, sem):
    wid = lax.axis_index("s") * _NC + lax.axis_index("c")
    base = wid * _BPW

    pltpu.sync_copy(target_hbm.at[pl.ds(base, _BPW)], tidx_s)
    pltpu.sync_copy(context_hbm.at[pl.ds(base, _BPW)], cidx_s)

    lane = lax.iota(jnp.int32, _L)

    for h in range(2):
        off = h * _HALF

        def fire(b, carry):
            tvec = tidx_s[pl.ds(off + b * _L, _L)]
            cvec = cidx_s[pl.ds(off + b * _L, _L)]
            for l in range(_L):
                it = tvec[l]
                ic = cvec[l]
                j = b * _L + l
                pltpu.async_copy(emb_hbm.at[pl.ds(it, 1), :],
                                 trows_v.at[pl.ds(j, 1), :], sem)
                pltpu.async_copy(outtab_hbm.at[pl.ds(ic, 1), :],
                                 crows_v.at[pl.ds(j, 1), :], sem)
            return carry

        lax.fori_loop(0, _HALF // _L, fire, 0)

        def drain(j, carry):
            pltpu.make_async_copy(emb_hbm.at[pl.ds(0, 1), :],
                                  trows_v.at[pl.ds(0, 1), :], sem).wait()
            pltpu.make_async_copy(outtab_hbm.at[pl.ds(0, 1), :],
                                  crows_v.at[pl.ds(0, 1), :], sem).wait()
            return carry

        lax.fori_loop(0, _HALF, drain, 0)

        def group(g, carry):
            row = g * _L + lane
            acc = jnp.zeros((_L,), jnp.float32)
            for d in range(DIM):
                col = jnp.full((_L,), d, jnp.int32)
                tv = plsc.load_gather(trows_v, [row, col])
                cv = plsc.load_gather(crows_v, [row, col])
                acc = acc + tv * cv
            res_v[pl.ds(off + g * _L, _L)] = acc
            return carry

        lax.fori_loop(0, _HALF // _L, group, 0)

    pltpu.sync_copy(res_v, out_hbm.at[pl.ds(base, _BPW)])


def kernel(target, context, embeddings, output):
    mesh = plsc.VectorSubcoreMesh(core_axis_name="c", subcore_axis_name="s")
    f = pl.kernel(
        _body,
        out_type=jax.ShapeDtypeStruct((BATCH,), jnp.float32),
        mesh=mesh,
        scratch_types=[
            pltpu.VMEM((_BPW,), jnp.int32),
            pltpu.VMEM((_BPW,), jnp.int32),
            pltpu.VMEM((_HALF, DIM), jnp.float32),
            pltpu.VMEM((_HALF, DIM), jnp.float32),
            pltpu.VMEM((_BPW,), jnp.float32),
            pltpu.SemaphoreType.DMA,
        ],
        compiler_params=pltpu.CompilerParams(needs_layout_passes=False),
    )
    return f(target.astype(jnp.int32), context.astype(jnp.int32),
             embeddings, output)
